# baseline (speedup 1.0000x reference)
import functools

import jax
import jax.numpy as jnp
from jax import lax
from jax.experimental import pallas as pl
from jax.experimental.pallas import tpu as pltpu
from jax.experimental.pallas import tpu_sc as plsc

NUM_EMB = 1_000_000
DIM = 64
B_TOTAL = 4096 * 200
NC, NS = 2, 16
NW = NC * NS
PER_W = B_TOTAL // NW
CHUNK = 128
K = 5
GROUP = CHUNK * K
NGROUPS = PER_W // GROUP
IDX_ROWS = PER_W // CHUNK


def _emb_body(words_hbm, weight_hbm, out_hbm, idx_v, rows_v, gsem, wsem):
    wid = lax.axis_index("s") * NC + lax.axis_index("c")
    base = wid * PER_W

    pltpu.sync_copy(words_hbm.at[wid], idx_v)

    def gather_copy(g, b, j):
        return pltpu.make_async_copy(
            weight_hbm.at[idx_v.at[g * K + j]],
            rows_v.at[b, pl.ds(j * CHUNK, CHUNK)],
            gsem.at[b],
        )

    def start_group(g, b):
        for j in range(K):
            gather_copy(g, b, j).start()

    def wait_group(g, b):
        for j in range(K):
            gather_copy(g, b, j).wait()

    def write_copy(g, b):
        return pltpu.make_async_copy(
            rows_v.at[b],
            out_hbm.at[pl.ds(base + g * GROUP, GROUP)],
            wsem.at[b],
        )

    start_group(0, 0)
    start_group(1, 1)

    def body(i, carry):
        g = 2 * i
        for b in (0, 1):
            wait_group(g + b, b)
            write_copy(g + b, b).start()
            write_copy(g + b, b).wait()
            start_group(g + b + 2, b)
        return carry

    lax.fori_loop(0, (NGROUPS - 2) // 2, body, 0)

    for b in (0, 1):
        g = NGROUPS - 2 + b
        wait_group(g, b)
        write_copy(g, b).start()
    for b in (0, 1):
        write_copy(NGROUPS - 2 + b, b).wait()


@functools.partial(jax.jit)
def _embedding_gather(words3d, weight):
    mesh = plsc.VectorSubcoreMesh(core_axis_name="c", subcore_axis_name="s")
    f = pl.kernel(
        _emb_body,
        out_type=jax.ShapeDtypeStruct((B_TOTAL, DIM), jnp.float32),
        mesh=mesh,
        scratch_types=[
            pltpu.VMEM((IDX_ROWS, CHUNK), jnp.int32),
            pltpu.VMEM((2, GROUP, DIM), jnp.float32),
            pltpu.SemaphoreType.DMA((2,)),
            pltpu.SemaphoreType.DMA((2,)),
        ],
        compiler_params=pltpu.CompilerParams(use_tc_tiling_on_sc=False),
    )
    return f(words3d, weight)


def kernel(words, weight):
    words3d = words.reshape(NW, IDX_ROWS, CHUNK).astype(jnp.int32)
    out = _embedding_gather(words3d, weight)
    return out.reshape(4096, 200, DIM)
